# baseline (speedup 1.0000x reference)
import functools

import jax
import jax.numpy as jnp
from jax import lax
from jax.experimental import pallas as pl
from jax.experimental.pallas import tpu as pltpu
from jax.experimental.pallas import tpu_sc as plsc

N = 4096
D = 256
P = 5532
R = 1000
OIM_SCALAR = 30.0

PP = 5632
RP = 1024

NW = 32
C = N // NW
L = 16

BT = 512
NB = N // BT


def _sc_body(roi_hbm, lut_hbm, rlab_hbm,
             feat_hbm, lr_hbm, fg_hbm, rm_hbm,
             roi_v, idx_v, lraw_v, feat_v, lr_v, fg_v, rm_v, sem, sem2):
    wid = lax.axis_index("s") * 2 + lax.axis_index("c")
    base = wid * C
    pltpu.sync_copy(roi_hbm.at[pl.ds(base, C)], roi_v)
    for j in range(C // L):
        rv = roi_v[pl.ds(j * L, L)]
        tg = rv - 1
        fg = tg >= 0
        safe = jnp.where(fg, tg, 0)
        idx_v[pl.ds(j * L, L)] = safe
        fg_v[pl.ds(j * L, L)] = jnp.where(fg, 1.0, 0.0).astype(jnp.float32)
    cp_feat = pltpu.async_copy(lut_hbm.at[idx_v], feat_v, sem)
    cp_lab = pltpu.async_copy(rlab_hbm.at[idx_v], lraw_v, sem2)
    cp_lab.wait()
    for j in range(C // L):
        lr = lraw_v[pl.ds(j * L, L)]
        fg = fg_v[pl.ds(j * L, L)] > 0.0
        rm = jnp.logical_and(fg, lr >= 0)
        lr_v[pl.ds(j * L, L)] = jnp.where(rm, lr, 0)
        rm_v[pl.ds(j * L, L)] = jnp.where(rm, 1.0, 0.0).astype(jnp.float32)
    cp_feat.wait()
    pltpu.sync_copy(feat_v, feat_hbm.at[pl.ds(base, C)])
    pltpu.sync_copy(lr_v, lr_hbm.at[pl.ds(base, C)])
    pltpu.sync_copy(fg_v, fg_hbm.at[pl.ds(base, C)])
    pltpu.sync_copy(rm_v, rm_hbm.at[pl.ds(base, C)])


_sc_gather = functools.partial(
    pl.kernel,
    out_type=[
        jax.ShapeDtypeStruct((N, D), jnp.float32),
        jax.ShapeDtypeStruct((N,), jnp.int32),
        jax.ShapeDtypeStruct((N,), jnp.float32),
        jax.ShapeDtypeStruct((N,), jnp.float32),
    ],
    mesh=plsc.VectorSubcoreMesh(core_axis_name="c", subcore_axis_name="s"),
    scratch_types=[
        pltpu.VMEM((C,), jnp.int32),
        pltpu.VMEM((C,), jnp.int32),
        pltpu.VMEM((C,), jnp.int32),
        pltpu.VMEM((C, D), jnp.float32),
        pltpu.VMEM((C,), jnp.int32),
        pltpu.VMEM((C,), jnp.float32),
        pltpu.VMEM((C,), jnp.float32),
        pltpu.SemaphoreType.DMA,
        pltpu.SemaphoreType.DMA,
    ],
)(_sc_body)


def _tc_body(x_ref, lut_ref, rlut_ref, rlab_ref, rlabt_ref, lr_ref, fg_ref,
             rm_ref, feat_ref, out_ref, acc_ref, hist_ref):
    i = pl.program_id(0)

    @pl.when(i == 0)
    def _init():
        for k in range(5):
            acc_ref[k] = 0.0
        for c in range(8):
            ii = lax.broadcasted_iota(jnp.int32, (P, 128), 1) + (c * 128)
            hc = jnp.sum(jnp.where(rlabt_ref[...] == ii, 1.0, 0.0),
                         axis=0, keepdims=True)
            hist_ref[0:1, c * 128:(c + 1) * 128] = hc

    x = x_ref[...]
    lr_col = lr_ref[...]
    rm = rm_ref[...]
    fg = fg_ref[...]

    xs = x * (OIM_SCALAR * 1.4426950408889634)

    l = lax.dot_general(xs, lut_ref[...], (((1,), (1,)), ((), ())),
                        preferred_element_type=jnp.float32)
    pos = rlab_ref[...] == lr_col
    neg = jnp.where(pos, -jnp.inf, l)
    m = jnp.max(neg, axis=1, keepdims=True)
    msafe = jnp.maximum(m, -1e37)
    En = jnp.exp2(neg - msafe)
    s = jnp.sum(En, axis=1, keepdims=True)
    lse2 = msafe + jnp.log2(s)
    v = lse2 - l
    per = jnp.maximum(v, 0.0) + jnp.log2(1.0 + jnp.exp2(0.0 - jnp.abs(v)))
    per = jnp.where(pos, per, 0.0)
    rowsum = jnp.sum(per, axis=1, keepdims=True)

    l2 = lax.dot_general(xs, rlut_ref[...], (((1,), (1,)), ((), ())),
                         preferred_element_type=jnp.float32)
    colr = lax.broadcasted_iota(jnp.int32, (BT, R), 1)
    mr = jnp.max(l2, axis=1, keepdims=True)
    sr = jnp.sum(jnp.exp2(l2 - mr), axis=1, keepdims=True)
    lser = mr + jnp.log2(sr)
    onehot = colr == lr_col
    pick = jnp.sum(jnp.where(onehot, l2, 0.0), axis=1, keepdims=True)
    reid_sum = jnp.sum((pick - lser) * rm)

    histb = hist_ref[0:1, 0:R]
    cnt = jnp.sum(jnp.where(onehot, histb, 0.0), axis=1, keepdims=True)
    rowl = rowsum / jnp.maximum(cnt, 1.0)
    inst = jnp.sum(rowl * rm)

    d2 = feat_ref[...] - x
    mse = jnp.sum(jnp.sum(d2 * d2, axis=1, keepdims=True) * fg)

    acc_ref[0] += inst
    acc_ref[1] += reid_sum
    acc_ref[2] += mse
    acc_ref[3] += jnp.sum(fg)
    acc_ref[4] += jnp.sum(rm)

    @pl.when(i == NB - 1)
    def _fin():
        fg_cnt = acc_ref[3]
        reid_cnt = acc_ref[4]
        ln2 = 0.6931471805599453
        total = (acc_ref[2] / (fg_cnt * float(D))
                 + (acc_ref[0] - acc_ref[1]) * ln2 / reid_cnt)
        out_ref[...] = jnp.full((1, 1), total, jnp.float32)


def kernel(inputs, roi_label, lut_instance, reid_lut, reid_labels):
    roi_label = roi_label.astype(jnp.int32)
    reid_labels = reid_labels.astype(jnp.int32)

    feat, label_reid, fg, rm = _sc_gather(roi_label, lut_instance, reid_labels)

    out = pl.pallas_call(
        _tc_body,
        grid=(NB,),
        in_specs=[
            pl.BlockSpec((BT, D), lambda i: (i, 0)),
            pl.BlockSpec((P, D), lambda i: (0, 0)),
            pl.BlockSpec((R, D), lambda i: (0, 0)),
            pl.BlockSpec((1, P), lambda i: (0, 0)),
            pl.BlockSpec((P, 1), lambda i: (0, 0)),
            pl.BlockSpec((BT, 1), lambda i: (i, 0)),
            pl.BlockSpec((BT, 1), lambda i: (i, 0)),
            pl.BlockSpec((BT, 1), lambda i: (i, 0)),
            pl.BlockSpec((BT, D), lambda i: (i, 0)),
        ],
        out_specs=pl.BlockSpec((1, 1), lambda i: (0, 0)),
        out_shape=jax.ShapeDtypeStruct((1, 1), jnp.float32),
        scratch_shapes=[pltpu.SMEM((8,), jnp.float32),
                        pltpu.VMEM((1, RP), jnp.float32)],
    )(inputs, lut_instance, reid_lut, reid_labels.reshape(1, P),
      reid_labels.reshape(P, 1),
      label_reid.reshape(N, 1), fg.reshape(N, 1), rm.reshape(N, 1), feat)

    return out[0, 0]


# speedup vs baseline: 1.0194x; 1.0194x over previous
import functools

import jax
import jax.numpy as jnp
from jax import lax
from jax.experimental import pallas as pl
from jax.experimental.pallas import tpu as pltpu
from jax.experimental.pallas import tpu_sc as plsc

N = 4096
D = 256
P = 5532
R = 1000
OIM_SCALAR = 30.0

PP = 5632
RP = 1024

NW = 32
C = N // NW
L = 16

BT = 512
NB = N // BT


def _sc_body(roi_hbm, lut_hbm, rlab_hbm,
             feat_hbm, lr_hbm, fg_hbm, rm_hbm,
             roi_v, idx_v, lraw_v, feat_v, lr_v, fg_v, rm_v, sem, sem2):
    wid = lax.axis_index("s") * 2 + lax.axis_index("c")
    base = wid * C
    pltpu.sync_copy(roi_hbm.at[pl.ds(base, C)], roi_v)
    for j in range(C // L):
        rv = roi_v[pl.ds(j * L, L)]
        tg = rv - 1
        fg = tg >= 0
        safe = jnp.where(fg, tg, 0)
        idx_v[pl.ds(j * L, L)] = safe
        fg_v[pl.ds(j * L, L)] = jnp.where(fg, 1.0, 0.0).astype(jnp.float32)
    cp_feat = pltpu.async_copy(lut_hbm.at[idx_v], feat_v, sem)
    cp_lab = pltpu.async_copy(rlab_hbm.at[idx_v], lraw_v, sem2)
    cp_lab.wait()
    for j in range(C // L):
        lr = lraw_v[pl.ds(j * L, L)]
        fg = fg_v[pl.ds(j * L, L)] > 0.0
        rm = jnp.logical_and(fg, lr >= 0)
        lr_v[pl.ds(j * L, L)] = jnp.where(rm, lr, 0)
        rm_v[pl.ds(j * L, L)] = jnp.where(rm, 1.0, 0.0).astype(jnp.float32)
    cp_feat.wait()
    pltpu.sync_copy(feat_v, feat_hbm.at[pl.ds(base, C)])
    pltpu.sync_copy(lr_v, lr_hbm.at[pl.ds(base, C)])
    pltpu.sync_copy(fg_v, fg_hbm.at[pl.ds(base, C)])
    pltpu.sync_copy(rm_v, rm_hbm.at[pl.ds(base, C)])


_sc_gather = functools.partial(
    pl.kernel,
    out_type=[
        jax.ShapeDtypeStruct((N, D), jnp.float32),
        jax.ShapeDtypeStruct((N,), jnp.int32),
        jax.ShapeDtypeStruct((N,), jnp.float32),
        jax.ShapeDtypeStruct((N,), jnp.float32),
    ],
    mesh=plsc.VectorSubcoreMesh(core_axis_name="c", subcore_axis_name="s"),
    scratch_types=[
        pltpu.VMEM((C,), jnp.int32),
        pltpu.VMEM((C,), jnp.int32),
        pltpu.VMEM((C,), jnp.int32),
        pltpu.VMEM((C, D), jnp.float32),
        pltpu.VMEM((C,), jnp.int32),
        pltpu.VMEM((C,), jnp.float32),
        pltpu.VMEM((C,), jnp.float32),
        pltpu.SemaphoreType.DMA,
        pltpu.SemaphoreType.DMA,
    ],
)(_sc_body)


def _tc_body(x_ref, lut_ref, rlut_ref, rlab_ref, lr_ref, fg_ref,
             rm_ref, feat_ref, out_ref, acc_ref):
    i = pl.program_id(0)

    @pl.when(i == 0)
    def _init():
        for k in range(5):
            acc_ref[k] = 0.0

    x = x_ref[...]
    lr_col = lr_ref[...]
    rm = rm_ref[...]
    fg = fg_ref[...]

    xs = x * (OIM_SCALAR * 1.4426950408889634)

    l = lax.dot_general(xs, lut_ref[...], (((1,), (1,)), ((), ())),
                        preferred_element_type=jnp.float32)
    pos = rlab_ref[...] == lr_col
    neg = jnp.where(pos, -jnp.inf, l)
    m = jnp.max(neg, axis=1, keepdims=True)
    msafe = jnp.maximum(m, -1e37)
    En = jnp.exp2(neg - msafe)
    s = jnp.sum(En, axis=1, keepdims=True)
    lse2 = msafe + jnp.log2(s)
    v = lse2 - l
    per = jnp.maximum(v, 0.0) + jnp.log2(1.0 + jnp.exp2(0.0 - jnp.abs(v)))
    per = jnp.where(pos, per, 0.0)
    rowsum = jnp.sum(per, axis=1, keepdims=True)

    l2 = lax.dot_general(xs, rlut_ref[...], (((1,), (1,)), ((), ())),
                         preferred_element_type=jnp.float32)
    colr = lax.broadcasted_iota(jnp.int32, (BT, R), 1)
    mr = jnp.max(l2, axis=1, keepdims=True)
    sr = jnp.sum(jnp.exp2(l2 - mr), axis=1, keepdims=True)
    lser = mr + jnp.log2(sr)
    onehot = colr == lr_col
    pick = jnp.sum(jnp.where(onehot, l2, 0.0), axis=1, keepdims=True)
    reid_sum = jnp.sum((pick - lser) * rm)

    cnt = jnp.sum(jnp.where(pos, 1.0, 0.0), axis=1, keepdims=True)
    rowl = rowsum / jnp.maximum(cnt, 1.0)
    inst = jnp.sum(rowl * rm)

    d2 = feat_ref[...] - x
    mse = jnp.sum(jnp.sum(d2 * d2, axis=1, keepdims=True) * fg)

    acc_ref[0] += inst
    acc_ref[1] += reid_sum
    acc_ref[2] += mse
    acc_ref[3] += jnp.sum(fg)
    acc_ref[4] += jnp.sum(rm)

    @pl.when(i == NB - 1)
    def _fin():
        fg_cnt = acc_ref[3]
        reid_cnt = acc_ref[4]
        ln2 = 0.6931471805599453
        total = (acc_ref[2] / (fg_cnt * float(D))
                 + (acc_ref[0] - acc_ref[1]) * ln2 / reid_cnt)
        out_ref[...] = jnp.full((1, 1), total, jnp.float32)


def kernel(inputs, roi_label, lut_instance, reid_lut, reid_labels):
    roi_label = roi_label.astype(jnp.int32)
    reid_labels = reid_labels.astype(jnp.int32)

    feat, label_reid, fg, rm = _sc_gather(roi_label, lut_instance, reid_labels)

    out = pl.pallas_call(
        _tc_body,
        grid=(NB,),
        in_specs=[
            pl.BlockSpec((BT, D), lambda i: (i, 0)),
            pl.BlockSpec((P, D), lambda i: (0, 0)),
            pl.BlockSpec((R, D), lambda i: (0, 0)),
            pl.BlockSpec((1, P), lambda i: (0, 0)),
            pl.BlockSpec((BT, 1), lambda i: (i, 0)),
            pl.BlockSpec((BT, 1), lambda i: (i, 0)),
            pl.BlockSpec((BT, 1), lambda i: (i, 0)),
            pl.BlockSpec((BT, D), lambda i: (i, 0)),
        ],
        out_specs=pl.BlockSpec((1, 1), lambda i: (0, 0)),
        out_shape=jax.ShapeDtypeStruct((1, 1), jnp.float32),
        scratch_shapes=[pltpu.SMEM((8,), jnp.float32)],
    )(inputs, lut_instance, reid_lut, reid_labels.reshape(1, P),
      label_reid.reshape(N, 1), fg.reshape(N, 1), rm.reshape(N, 1), feat)

    return out[0, 0]


# speedup vs baseline: 1.1690x; 1.1468x over previous
import functools

import jax
import jax.numpy as jnp
from jax import lax
from jax.experimental import pallas as pl
from jax.experimental.pallas import tpu as pltpu
from jax.experimental.pallas import tpu_sc as plsc

N = 4096
D = 256
P = 5532
R = 1000
OIM_SCALAR = 30.0

PP = 5632
RP = 1024

NW = 32
C = N // NW
L = 16

BT = 512
NB = N // BT


def _sc_body(roi_hbm, lut_hbm, rlab_hbm,
             feat_hbm, lr_hbm, fg_hbm, rm_hbm,
             roi_v, idx_v, lraw_v, feat_v, lr_v, fg_v, rm_v, sem, sem2):
    wid = lax.axis_index("s") * 2 + lax.axis_index("c")
    base = wid * C
    pltpu.sync_copy(roi_hbm.at[pl.ds(base, C)], roi_v)
    for j in range(C // L):
        rv = roi_v[pl.ds(j * L, L)]
        tg = rv - 1
        fg = tg >= 0
        safe = jnp.where(fg, tg, 0)
        idx_v[pl.ds(j * L, L)] = safe
        fg_v[pl.ds(j * L, L)] = jnp.where(fg, 1.0, 0.0).astype(jnp.float32)
    cp_feat = pltpu.async_copy(lut_hbm.at[idx_v], feat_v, sem)
    cp_lab = pltpu.async_copy(rlab_hbm.at[idx_v], lraw_v, sem2)
    cp_lab.wait()
    for j in range(C // L):
        lr = lraw_v[pl.ds(j * L, L)]
        fg = fg_v[pl.ds(j * L, L)] > 0.0
        rm = jnp.logical_and(fg, lr >= 0)
        lr_v[pl.ds(j * L, L)] = jnp.where(rm, lr, 0)
        rm_v[pl.ds(j * L, L)] = jnp.where(rm, 1.0, 0.0).astype(jnp.float32)
    cp_feat.wait()
    pltpu.sync_copy(feat_v, feat_hbm.at[pl.ds(base, C)])
    pltpu.sync_copy(lr_v, lr_hbm.at[pl.ds(base, C)])
    pltpu.sync_copy(fg_v, fg_hbm.at[pl.ds(base, C)])
    pltpu.sync_copy(rm_v, rm_hbm.at[pl.ds(base, C)])


_sc_gather = functools.partial(
    pl.kernel,
    out_type=[
        jax.ShapeDtypeStruct((N, D), jnp.float32),
        jax.ShapeDtypeStruct((N,), jnp.int32),
        jax.ShapeDtypeStruct((N,), jnp.float32),
        jax.ShapeDtypeStruct((N,), jnp.float32),
    ],
    mesh=plsc.VectorSubcoreMesh(core_axis_name="c", subcore_axis_name="s"),
    scratch_types=[
        pltpu.VMEM((C,), jnp.int32),
        pltpu.VMEM((C,), jnp.int32),
        pltpu.VMEM((C,), jnp.int32),
        pltpu.VMEM((C, D), jnp.float32),
        pltpu.VMEM((C,), jnp.int32),
        pltpu.VMEM((C,), jnp.float32),
        pltpu.VMEM((C,), jnp.float32),
        pltpu.SemaphoreType.DMA,
        pltpu.SemaphoreType.DMA,
    ],
)(_sc_body)


def _tc_body(x_ref, lut_ref, rlut_ref, rlab_ref, lr_ref, fg_ref,
             rm_ref, feat_ref, out_ref, acc_ref):
    i = pl.program_id(0)

    @pl.when(i == 0)
    def _init():
        for k in range(5):
            acc_ref[k] = 0.0

    x = x_ref[...]
    lr_col = lr_ref[...]
    rm = rm_ref[...]
    fg = fg_ref[...]

    xs = x * (OIM_SCALAR * 1.4426950408889634)

    l = lax.dot_general(xs, lut_ref[...], (((1,), (1,)), ((), ())),
                        preferred_element_type=jnp.float32)
    pos = rlab_ref[...] == lr_col
    m = jnp.max(l, axis=1, keepdims=True)
    t = l - m
    E = jnp.exp2(t)
    s = jnp.sum(jnp.where(pos, 0.0, E), axis=1, keepdims=True)
    perm = jnp.log2(E + s) - t
    rowsum = jnp.sum(jnp.where(pos, perm, 0.0), axis=1, keepdims=True)

    l2 = lax.dot_general(xs, rlut_ref[...], (((1,), (1,)), ((), ())),
                         preferred_element_type=jnp.float32)
    colr = lax.broadcasted_iota(jnp.int32, (BT, R), 1)
    mr = jnp.max(l2, axis=1, keepdims=True)
    sr = jnp.sum(jnp.exp2(l2 - mr), axis=1, keepdims=True)
    lser = mr + jnp.log2(sr)
    onehot = colr == lr_col
    pick = jnp.sum(jnp.where(onehot, l2, 0.0), axis=1, keepdims=True)
    reid_sum = jnp.sum((pick - lser) * rm)

    cnt = jnp.sum(jnp.where(pos, 1.0, 0.0), axis=1, keepdims=True)
    rowl = rowsum / jnp.maximum(cnt, 1.0)
    inst = jnp.sum(rowl * rm)

    d2 = feat_ref[...] - x
    mse = jnp.sum(jnp.sum(d2 * d2, axis=1, keepdims=True) * fg)

    acc_ref[0] += inst
    acc_ref[1] += reid_sum
    acc_ref[2] += mse
    acc_ref[3] += jnp.sum(fg)
    acc_ref[4] += jnp.sum(rm)

    @pl.when(i == NB - 1)
    def _fin():
        fg_cnt = acc_ref[3]
        reid_cnt = acc_ref[4]
        ln2 = 0.6931471805599453
        total = (acc_ref[2] / (fg_cnt * float(D))
                 + (acc_ref[0] - acc_ref[1]) * ln2 / reid_cnt)
        out_ref[...] = jnp.full((1, 1), total, jnp.float32)


def kernel(inputs, roi_label, lut_instance, reid_lut, reid_labels):
    roi_label = roi_label.astype(jnp.int32)
    reid_labels = reid_labels.astype(jnp.int32)

    feat, label_reid, fg, rm = _sc_gather(roi_label, lut_instance, reid_labels)

    out = pl.pallas_call(
        _tc_body,
        grid=(NB,),
        in_specs=[
            pl.BlockSpec((BT, D), lambda i: (i, 0)),
            pl.BlockSpec((P, D), lambda i: (0, 0)),
            pl.BlockSpec((R, D), lambda i: (0, 0)),
            pl.BlockSpec((1, P), lambda i: (0, 0)),
            pl.BlockSpec((BT, 1), lambda i: (i, 0)),
            pl.BlockSpec((BT, 1), lambda i: (i, 0)),
            pl.BlockSpec((BT, 1), lambda i: (i, 0)),
            pl.BlockSpec((BT, D), lambda i: (i, 0)),
        ],
        out_specs=pl.BlockSpec((1, 1), lambda i: (0, 0)),
        out_shape=jax.ShapeDtypeStruct((1, 1), jnp.float32),
        scratch_shapes=[pltpu.SMEM((8,), jnp.float32)],
    )(inputs, lut_instance, reid_lut, reid_labels.reshape(1, P),
      label_reid.reshape(N, 1), fg.reshape(N, 1), rm.reshape(N, 1), feat)

    return out[0, 0]


# speedup vs baseline: 1.1873x; 1.0157x over previous
import functools

import jax
import jax.numpy as jnp
from jax import lax
from jax.experimental import pallas as pl
from jax.experimental.pallas import tpu as pltpu
from jax.experimental.pallas import tpu_sc as plsc

N = 4096
D = 256
P = 5532
R = 1000
OIM_SCALAR = 30.0

PP = 5632
RP = 1024

NW = 32
C = N // NW
L = 16

BT = 1024
NB = N // BT


def _sc_body(roi_hbm, lut_hbm, rlab_hbm,
             feat_hbm, lr_hbm, fg_hbm, rm_hbm,
             roi_v, idx_v, lraw_v, feat_v, lr_v, fg_v, rm_v, sem, sem2):
    wid = lax.axis_index("s") * 2 + lax.axis_index("c")
    base = wid * C
    pltpu.sync_copy(roi_hbm.at[pl.ds(base, C)], roi_v)
    for j in range(C // L):
        rv = roi_v[pl.ds(j * L, L)]
        tg = rv - 1
        fg = tg >= 0
        safe = jnp.where(fg, tg, 0)
        idx_v[pl.ds(j * L, L)] = safe
        fg_v[pl.ds(j * L, L)] = jnp.where(fg, 1.0, 0.0).astype(jnp.float32)
    cp_feat = pltpu.async_copy(lut_hbm.at[idx_v], feat_v, sem)
    cp_lab = pltpu.async_copy(rlab_hbm.at[idx_v], lraw_v, sem2)
    cp_lab.wait()
    for j in range(C // L):
        lr = lraw_v[pl.ds(j * L, L)]
        fg = fg_v[pl.ds(j * L, L)] > 0.0
        rm = jnp.logical_and(fg, lr >= 0)
        lr_v[pl.ds(j * L, L)] = jnp.where(rm, lr, 0)
        rm_v[pl.ds(j * L, L)] = jnp.where(rm, 1.0, 0.0).astype(jnp.float32)
    cp_feat.wait()
    pltpu.sync_copy(feat_v, feat_hbm.at[pl.ds(base, C)])
    pltpu.sync_copy(lr_v, lr_hbm.at[pl.ds(base, C)])
    pltpu.sync_copy(fg_v, fg_hbm.at[pl.ds(base, C)])
    pltpu.sync_copy(rm_v, rm_hbm.at[pl.ds(base, C)])


_sc_gather = functools.partial(
    pl.kernel,
    out_type=[
        jax.ShapeDtypeStruct((N, D), jnp.float32),
        jax.ShapeDtypeStruct((N,), jnp.int32),
        jax.ShapeDtypeStruct((N,), jnp.float32),
        jax.ShapeDtypeStruct((N,), jnp.float32),
    ],
    mesh=plsc.VectorSubcoreMesh(core_axis_name="c", subcore_axis_name="s"),
    scratch_types=[
        pltpu.VMEM((C,), jnp.int32),
        pltpu.VMEM((C,), jnp.int32),
        pltpu.VMEM((C,), jnp.int32),
        pltpu.VMEM((C, D), jnp.float32),
        pltpu.VMEM((C,), jnp.int32),
        pltpu.VMEM((C,), jnp.float32),
        pltpu.VMEM((C,), jnp.float32),
        pltpu.SemaphoreType.DMA,
        pltpu.SemaphoreType.DMA,
    ],
)(_sc_body)


def _tc_body(x_ref, lut_ref, rlut_ref, rlab_ref, lr_ref, fg_ref,
             rm_ref, feat_ref, out_ref, acc_ref):
    i = pl.program_id(0)

    @pl.when(i == 0)
    def _init():
        for k in range(5):
            acc_ref[k] = 0.0

    x = x_ref[...]
    lr_col = lr_ref[...]
    rm = rm_ref[...]
    fg = fg_ref[...]

    xs = x * (OIM_SCALAR * 1.4426950408889634)

    l = lax.dot_general(xs, lut_ref[...], (((1,), (1,)), ((), ())),
                        preferred_element_type=jnp.float32)
    pos = rlab_ref[...] == lr_col
    m = jnp.max(l, axis=1, keepdims=True)
    t = l - m
    E = jnp.exp2(t)
    s = jnp.sum(jnp.where(pos, 0.0, E), axis=1, keepdims=True)
    perm = jnp.log2(E + s) - t
    rowsum = jnp.sum(jnp.where(pos, perm, 0.0), axis=1, keepdims=True)

    l2 = lax.dot_general(xs, rlut_ref[...], (((1,), (1,)), ((), ())),
                         preferred_element_type=jnp.float32)
    colr = lax.broadcasted_iota(jnp.int32, (BT, R), 1)
    mr = jnp.max(l2, axis=1, keepdims=True)
    sr = jnp.sum(jnp.exp2(l2 - mr), axis=1, keepdims=True)
    lser = mr + jnp.log2(sr)
    onehot = colr == lr_col
    pick = jnp.sum(jnp.where(onehot, l2, 0.0), axis=1, keepdims=True)
    reid_sum = jnp.sum((pick - lser) * rm)

    cnt = jnp.sum(jnp.where(pos, 1.0, 0.0), axis=1, keepdims=True)
    rowl = rowsum / jnp.maximum(cnt, 1.0)
    inst = jnp.sum(rowl * rm)

    d2 = feat_ref[...] - x
    mse = jnp.sum(jnp.sum(d2 * d2, axis=1, keepdims=True) * fg)

    acc_ref[0] += inst
    acc_ref[1] += reid_sum
    acc_ref[2] += mse
    acc_ref[3] += jnp.sum(fg)
    acc_ref[4] += jnp.sum(rm)

    @pl.when(i == NB - 1)
    def _fin():
        fg_cnt = acc_ref[3]
        reid_cnt = acc_ref[4]
        ln2 = 0.6931471805599453
        total = (acc_ref[2] / (fg_cnt * float(D))
                 + (acc_ref[0] - acc_ref[1]) * ln2 / reid_cnt)
        out_ref[...] = jnp.full((1, 1), total, jnp.float32)


def kernel(inputs, roi_label, lut_instance, reid_lut, reid_labels):
    roi_label = roi_label.astype(jnp.int32)
    reid_labels = reid_labels.astype(jnp.int32)

    feat, label_reid, fg, rm = _sc_gather(roi_label, lut_instance, reid_labels)

    out = pl.pallas_call(
        _tc_body,
        grid=(NB,),
        in_specs=[
            pl.BlockSpec((BT, D), lambda i: (i, 0)),
            pl.BlockSpec((P, D), lambda i: (0, 0)),
            pl.BlockSpec((R, D), lambda i: (0, 0)),
            pl.BlockSpec((1, P), lambda i: (0, 0)),
            pl.BlockSpec((BT, 1), lambda i: (i, 0)),
            pl.BlockSpec((BT, 1), lambda i: (i, 0)),
            pl.BlockSpec((BT, 1), lambda i: (i, 0)),
            pl.BlockSpec((BT, D), lambda i: (i, 0)),
        ],
        out_specs=pl.BlockSpec((1, 1), lambda i: (0, 0)),
        out_shape=jax.ShapeDtypeStruct((1, 1), jnp.float32),
        scratch_shapes=[pltpu.SMEM((8,), jnp.float32)],
    )(inputs, lut_instance, reid_lut, reid_labels.reshape(1, P),
      label_reid.reshape(N, 1), fg.reshape(N, 1), rm.reshape(N, 1), feat)

    return out[0, 0]
